# baseline (speedup 1.0000x reference)
import jax
import jax.numpy as jnp
import numpy as np
from jax.experimental import pallas as pl
from jax.experimental.pallas import tpu as pltpu

N_CHANNELS = 3
BLOCK_N = 1024

_LOG2E = float(np.log2(np.e))
_KQ = -0.5 * _LOG2E
_C2 = float(np.log2(_LOG2E / 2.0))
_COS_COEF = (1.0, -19.739208, 64.939384, -85.45664, 60.24202,
             -26.404266, 7.799566, -1.4530462)


def _coef_body(pr_ref, w8_ref, aux_ref):
    meanx = pr_ref[0:1, :]
    meany = pr_ref[1:2, :]
    m00 = pr_ref[2:3, :]
    m01 = pr_ref[3:4, :]
    m10 = pr_ref[4:5, :]
    m11 = pr_ref[5:6, :]
    rot = pr_ref[6:7, :]
    freq = pr_ref[7:8, :]
    off = pr_ref[8:9, :]
    ftp = pr_ref[9:10, :]
    logw = pr_ref[10:11, :]

    d0 = -(m00 * meanx + m01 * meany)
    d1 = -(m10 * meanx + m11 * meany)
    kq = jnp.float32(_KQ)
    qc_x2 = kq * (m00 * m00 + m10 * m10)
    qc_xy = (2.0 * kq) * (m00 * m01 + m10 * m11)
    qc_y2 = kq * (m01 * m01 + m11 * m11)
    qc_x = (2.0 * kq) * (m00 * d0 + m10 * d1)
    qc_y = (2.0 * kq) * (m01 * d0 + m11 * d1)
    qc_1 = kq * (d0 * d0 + d1 * d1)

    c = jnp.cos(rot)
    s = jnp.sin(rot)
    f2 = 2.0 * freq
    fa = f2 * c
    fb = f2 * s
    fc = off * (1.0 / np.pi) - (fa * meanx + fb * meany)

    zero = jnp.zeros_like(m00)
    qhalf = jnp.concatenate(
        [qc_x2, qc_xy, qc_y2, qc_x, qc_y, qc_1, zero, zero], axis=0)
    vhalf = jnp.concatenate(
        [zero, zero, zero, fa, fb, fc, zero, zero], axis=0)
    w8_ref[:, :] = jnp.concatenate([qhalf, vhalf], axis=1)

    hiw = 0.5 * jnp.exp(-2.0 * logw)
    aux_ref[:, :] = jnp.concatenate(
        [hiw, hiw + 1e-12, jnp.exp(ftp), zero, zero, zero, zero, zero],
        axis=0)


def _feat_body(x_ref, z_ref):
    x0 = x_ref[:, 0:1]
    x1 = x_ref[:, 1:2]
    one = jnp.ones_like(x0)
    z_ref[:, :] = jnp.concatenate(
        [x0 * x0, x0 * x1, x1 * x1, x0, x1, one, one, one], axis=1)


def _main_body(z_ref, w8_ref, aux_ref, col_ref, out_ref):
    w = aux_ref.shape[1]
    hiw = aux_ref[0:1, :]
    hw_eps = aux_ref[1:2, :]
    p = aux_ref[2:3, :]

    nv = jnp.dot(z_ref[:, :], w8_ref[:, :],
                 precision=jax.lax.Precision.HIGHEST,
                 preferred_element_type=jnp.float32)
    nqs = nv[:, :w]
    v = nv[:, w:]

    r = jax.lax.round(v, jax.lax.RoundingMethod.TO_NEAREST_EVEN)
    sf = v - r
    t = sf * sf
    ct = jnp.float32(_COS_COEF[7])
    for k in (6, 5, 4, 3, 2, 1, 0):
        ct = ct * t + jnp.float32(_COS_COEF[k])
    base = jnp.maximum(hw_eps - ct * hiw, 1e-12)
    e1 = jnp.exp2(p * jnp.log2(base) + _C2)
    vals = jnp.exp2(nqs - e1)

    out_ref[:, :] = jnp.dot(vals, col_ref[:, :],
                            preferred_element_type=jnp.float32)


@jax.jit
def kernel(x, gaussian_means, gaussian_mats, subgaussian_frequency,
           subgaussian_offset, subgaussian_flat_top_power,
           subgaussian_width, subgaussian_rotation, colors):
    n_pix = x.shape[0]
    w = gaussian_means.shape[0]

    params = jnp.concatenate([
        gaussian_means[:, 0][None, :],
        gaussian_means[:, 1][None, :],
        gaussian_mats[:, 0, 0][None, :],
        gaussian_mats[:, 0, 1][None, :],
        gaussian_mats[:, 1, 0][None, :],
        gaussian_mats[:, 1, 1][None, :],
        subgaussian_rotation.T,
        subgaussian_frequency.T,
        subgaussian_offset.T,
        subgaussian_flat_top_power.T,
        subgaussian_width.T,
        jnp.zeros((5, w), jnp.float32),
    ], axis=0)

    w8, aux = pl.pallas_call(
        _coef_body,
        in_specs=[pl.BlockSpec((16, w), lambda: (0, 0))],
        out_specs=[
            pl.BlockSpec((8, 2 * w), lambda: (0, 0)),
            pl.BlockSpec((8, w), lambda: (0, 0)),
        ],
        out_shape=[
            jax.ShapeDtypeStruct((8, 2 * w), jnp.float32),
            jax.ShapeDtypeStruct((8, w), jnp.float32),
        ],
    )(params)

    z = pl.pallas_call(
        _feat_body,
        grid=(n_pix // BLOCK_N,),
        in_specs=[pl.BlockSpec((BLOCK_N, 2), lambda i: (i, 0))],
        out_specs=pl.BlockSpec((BLOCK_N, 8), lambda i: (i, 0)),
        out_shape=jax.ShapeDtypeStruct((n_pix, 8), jnp.float32),
        compiler_params=pltpu.CompilerParams(
            dimension_semantics=("parallel",),
        ),
    )(x)

    return pl.pallas_call(
        _main_body,
        grid=(n_pix // BLOCK_N,),
        in_specs=[
            pl.BlockSpec((BLOCK_N, 8), lambda i: (i, 0)),
            pl.BlockSpec((8, 2 * w), lambda i: (0, 0)),
            pl.BlockSpec((8, w), lambda i: (0, 0)),
            pl.BlockSpec((w, N_CHANNELS), lambda i: (0, 0)),
        ],
        out_specs=pl.BlockSpec((BLOCK_N, N_CHANNELS), lambda i: (i, 0)),
        out_shape=jax.ShapeDtypeStruct((n_pix, N_CHANNELS), jnp.float32),
        compiler_params=pltpu.CompilerParams(
            dimension_semantics=("parallel",),
        ),
    )(z, w8, aux, colors)


# speedup vs baseline: 1.6280x; 1.6280x over previous
import jax
import jax.numpy as jnp
import numpy as np
from jax.experimental import pallas as pl
from jax.experimental.pallas import tpu as pltpu

N_CHANNELS = 3
BLOCK_N = 1024

_LOG2E = float(np.log2(np.e))
_KQ = -0.5 * _LOG2E
_C2 = float(np.log2(_LOG2E / 2.0))
_COS_COEF = (1.0, -19.739202, 64.93908, -85.4497, 60.16561,
             -25.964163, 6.5281506)


def _coef_body(pr_ref, mu_ref, mats_ref, col_ref, w6_ref, aux_ref, cs_ref):
    meanx = pr_ref[0:1, :]
    meany = pr_ref[1:2, :]
    m00 = pr_ref[2:3, :]
    m01 = pr_ref[3:4, :]
    m10 = pr_ref[4:5, :]
    m11 = pr_ref[5:6, :]
    rot = pr_ref[6:7, :]
    freq = pr_ref[7:8, :]
    off = pr_ref[8:9, :]
    ftp = pr_ref[9:10, :]
    logw = pr_ref[10:11, :]

    d0 = -(m00 * meanx + m01 * meany)
    d1 = -(m10 * meanx + m11 * meany)
    kq = jnp.float32(_KQ)
    qc_x2 = kq * (m00 * m00 + m10 * m10)
    qc_y2 = kq * (m01 * m01 + m11 * m11)
    qc_xy = (2.0 * kq) * (m00 * m01 + m10 * m11)
    qc_x = (2.0 * kq) * (m00 * d0 + m10 * d1)
    qc_y = (2.0 * kq) * (m01 * d0 + m11 * d1)

    c = jnp.cos(rot)
    s = jnp.sin(rot)
    f2 = 2.0 * freq
    fa = f2 * c
    fb = f2 * s
    fc = off * (1.0 / np.pi) - (fa * meanx + fb * meany)

    zero = jnp.zeros_like(m00)
    w6_ref[:, :] = jnp.concatenate(
        [qc_x2, qc_y2, qc_xy, qc_x, qc_y, zero, zero, zero], axis=0)

    hiw = 0.5 * jnp.exp(-2.0 * logw)
    aux_ref[:, :] = jnp.concatenate(
        [hiw, hiw + 1e-12, jnp.exp(ftp), fa, fb, fc, zero, zero], axis=0)

    mx = mu_ref[:, 0:1]
    my = mu_ref[:, 1:2]
    e0 = -(mats_ref[:, 0:1] * mx + mats_ref[:, 1:2] * my)
    e1 = -(mats_ref[:, 2:3] * mx + mats_ref[:, 3:4] * my)
    qc1 = kq * (e0 * e0 + e1 * e1)
    cs_ref[:, :] = col_ref[:, :] * jnp.exp2(qc1)


def _main_body(x_ref, w6_ref, aux_ref, cs_ref, out_ref):
    hiw = aux_ref[0:1, :]
    hw_eps = aux_ref[1:2, :]
    p = aux_ref[2:3, :]
    fa = aux_ref[3:4, :]
    fb = aux_ref[4:5, :]
    fc = aux_ref[5:6, :]

    xb = x_ref[:, :]
    x0 = x_ref[:, 0:1]
    x1 = x_ref[:, 1:2]
    feats = jnp.concatenate([xb * xb, x0 * x1, xb], axis=1)
    nqs = jnp.dot(feats, w6_ref[0:5, :],
                  precision=jax.lax.Precision.HIGHEST,
                  preferred_element_type=jnp.float32)

    v = fa * x0 + (fb * x1 + fc)
    r = jax.lax.round(v, jax.lax.RoundingMethod.TO_NEAREST_EVEN)
    sf = v - r
    t = sf * sf
    ct = jnp.float32(_COS_COEF[6])
    for k in (5, 4, 3, 2, 1, 0):
        ct = ct * t + jnp.float32(_COS_COEF[k])
    base = jnp.maximum(hw_eps - ct * hiw, 1e-12)
    e1 = jnp.exp2(p * jnp.log2(base) + _C2)
    vals = jnp.exp2(nqs - e1)

    out_ref[:, :] = jnp.dot(vals, cs_ref[:, :],
                            preferred_element_type=jnp.float32)


@jax.jit
def kernel(x, gaussian_means, gaussian_mats, subgaussian_frequency,
           subgaussian_offset, subgaussian_flat_top_power,
           subgaussian_width, subgaussian_rotation, colors):
    n_pix = x.shape[0]
    w = gaussian_means.shape[0]

    params = jnp.concatenate([
        gaussian_means[:, 0][None, :],
        gaussian_means[:, 1][None, :],
        gaussian_mats[:, 0, 0][None, :],
        gaussian_mats[:, 0, 1][None, :],
        gaussian_mats[:, 1, 0][None, :],
        gaussian_mats[:, 1, 1][None, :],
        subgaussian_rotation.T,
        subgaussian_frequency.T,
        subgaussian_offset.T,
        subgaussian_flat_top_power.T,
        subgaussian_width.T,
        jnp.zeros((5, w), jnp.float32),
    ], axis=0)
    mats4 = gaussian_mats.reshape(w, 4)

    w6, aux, colscaled = pl.pallas_call(
        _coef_body,
        in_specs=[
            pl.BlockSpec((16, w), lambda: (0, 0)),
            pl.BlockSpec((w, 2), lambda: (0, 0)),
            pl.BlockSpec((w, 4), lambda: (0, 0)),
            pl.BlockSpec((w, N_CHANNELS), lambda: (0, 0)),
        ],
        out_specs=[
            pl.BlockSpec((8, w), lambda: (0, 0)),
            pl.BlockSpec((8, w), lambda: (0, 0)),
            pl.BlockSpec((w, N_CHANNELS), lambda: (0, 0)),
        ],
        out_shape=[
            jax.ShapeDtypeStruct((8, w), jnp.float32),
            jax.ShapeDtypeStruct((8, w), jnp.float32),
            jax.ShapeDtypeStruct((w, N_CHANNELS), jnp.float32),
        ],
    )(params, gaussian_means, mats4, colors)

    return pl.pallas_call(
        _main_body,
        grid=(n_pix // BLOCK_N,),
        in_specs=[
            pl.BlockSpec((BLOCK_N, 2), lambda i: (i, 0)),
            pl.BlockSpec((8, w), lambda i: (0, 0)),
            pl.BlockSpec((8, w), lambda i: (0, 0)),
            pl.BlockSpec((w, N_CHANNELS), lambda i: (0, 0)),
        ],
        out_specs=pl.BlockSpec((BLOCK_N, N_CHANNELS), lambda i: (i, 0)),
        out_shape=jax.ShapeDtypeStruct((n_pix, N_CHANNELS), jnp.float32),
        compiler_params=pltpu.CompilerParams(
            dimension_semantics=("parallel",),
        ),
    )(x, w6, aux, colscaled)


# speedup vs baseline: 1.8347x; 1.1270x over previous
import jax
import jax.numpy as jnp
import numpy as np
from jax.experimental import pallas as pl
from jax.experimental.pallas import tpu as pltpu

N_CHANNELS = 3
BLOCK_N = 2048

_LOG2E = float(np.log2(np.e))
_SQH = float(np.sqrt(0.5 * _LOG2E))
_C2 = float(np.log2(_LOG2E / 2.0))
_COS_COEF = (1.0, -19.739208, 64.939384, -85.45664, 60.24202,
             -26.404266, 7.799566, -1.4530462)


def _body(x_ref, pr_ref, col_ref, out_ref):
    x0 = x_ref[:, 0:1]
    x1 = x_ref[:, 1:2]

    meanx = pr_ref[0:1, :]
    meany = pr_ref[1:2, :]
    m00 = pr_ref[2:3, :]
    m01 = pr_ref[3:4, :]
    m10 = pr_ref[4:5, :]
    m11 = pr_ref[5:6, :]
    rot = pr_ref[6:7, :]
    freq = pr_ref[7:8, :]
    off = pr_ref[8:9, :]
    ftp = pr_ref[9:10, :]
    logw = pr_ref[10:11, :]

    c = jnp.cos(rot)
    s = jnp.sin(rot)
    a0 = _SQH * m00
    b0 = _SQH * m01
    c0 = -(a0 * meanx + b0 * meany)
    a1 = _SQH * m10
    b1 = _SQH * m11
    c1 = -(a1 * meanx + b1 * meany)
    f2 = 2.0 * freq
    fa = f2 * c
    fb = f2 * s
    fc = off * (1.0 / np.pi) - (fa * meanx + fb * meany)
    half_inv_w2 = 0.5 * jnp.exp(-2.0 * logw)
    hw_eps = half_inv_w2 + 1e-12
    p = jnp.exp(ftp)

    u0 = a0 * x0 + (b0 * x1 + c0)
    u1 = a1 * x0 + (b1 * x1 + c1)
    v = fa * x0 + (fb * x1 + fc)

    r = jax.lax.round(v, jax.lax.RoundingMethod.TO_NEAREST_EVEN)
    sf = v - r
    t = sf * sf
    ct = jnp.float32(_COS_COEF[7])
    for k in (6, 5, 4, 3, 2, 1, 0):
        ct = ct * t + jnp.float32(_COS_COEF[k])
    base = jnp.maximum(hw_eps - ct * half_inv_w2, 1e-12)
    e1 = jnp.exp2(p * jnp.log2(base) + _C2)
    vals = jnp.exp2(-(u0 * u0 + (u1 * u1 + e1)))

    out_ref[:, :] = jnp.dot(vals, col_ref[:, :],
                            preferred_element_type=jnp.float32)


@jax.jit
def kernel(x, gaussian_means, gaussian_mats, subgaussian_frequency,
           subgaussian_offset, subgaussian_flat_top_power,
           subgaussian_width, subgaussian_rotation, colors):
    n_pix = x.shape[0]
    w = gaussian_means.shape[0]

    params = jnp.concatenate([
        gaussian_means[:, 0][None, :],
        gaussian_means[:, 1][None, :],
        gaussian_mats[:, 0, 0][None, :],
        gaussian_mats[:, 0, 1][None, :],
        gaussian_mats[:, 1, 0][None, :],
        gaussian_mats[:, 1, 1][None, :],
        subgaussian_rotation.T,
        subgaussian_frequency.T,
        subgaussian_offset.T,
        subgaussian_flat_top_power.T,
        subgaussian_width.T,
        jnp.zeros((5, w), jnp.float32),
    ], axis=0)

    grid = (n_pix // BLOCK_N,)
    return pl.pallas_call(
        _body,
        grid=grid,
        in_specs=[
            pl.BlockSpec((BLOCK_N, 2), lambda i: (i, 0)),
            pl.BlockSpec((16, w), lambda i: (0, 0)),
            pl.BlockSpec((w, N_CHANNELS), lambda i: (0, 0)),
        ],
        out_specs=pl.BlockSpec((BLOCK_N, N_CHANNELS), lambda i: (i, 0)),
        out_shape=jax.ShapeDtypeStruct((n_pix, N_CHANNELS), jnp.float32),
        compiler_params=pltpu.CompilerParams(
            dimension_semantics=("parallel",),
        ),
    )(x, params, colors)


# speedup vs baseline: 1.8675x; 1.0178x over previous
import jax
import jax.numpy as jnp
import numpy as np
from jax.experimental import pallas as pl
from jax.experimental.pallas import tpu as pltpu

N_CHANNELS = 3
BLOCK_N = 4096

_LOG2E = float(np.log2(np.e))
_SQH = float(np.sqrt(0.5 * _LOG2E))
_C2 = float(np.log2(_LOG2E / 2.0))
_COS_COEF = (1.0, -19.739208, 64.939384, -85.45664, 60.24202,
             -26.404266, 7.799566, -1.4530462)


def _body(x_ref, pr_ref, col_ref, out_ref):
    x0 = x_ref[:, 0:1]
    x1 = x_ref[:, 1:2]

    meanx = pr_ref[0:1, :]
    meany = pr_ref[1:2, :]
    m00 = pr_ref[2:3, :]
    m01 = pr_ref[3:4, :]
    m10 = pr_ref[4:5, :]
    m11 = pr_ref[5:6, :]
    rot = pr_ref[6:7, :]
    freq = pr_ref[7:8, :]
    off = pr_ref[8:9, :]
    ftp = pr_ref[9:10, :]
    logw = pr_ref[10:11, :]

    c = jnp.cos(rot)
    s = jnp.sin(rot)
    a0 = _SQH * m00
    b0 = _SQH * m01
    c0 = -(a0 * meanx + b0 * meany)
    a1 = _SQH * m10
    b1 = _SQH * m11
    c1 = -(a1 * meanx + b1 * meany)
    f2 = 2.0 * freq
    fa = f2 * c
    fb = f2 * s
    fc = off * (1.0 / np.pi) - (fa * meanx + fb * meany)
    half_inv_w2 = 0.5 * jnp.exp(-2.0 * logw)
    hw_eps = half_inv_w2 + 1e-12
    p = jnp.exp(ftp)

    u0 = a0 * x0 + (b0 * x1 + c0)
    u1 = a1 * x0 + (b1 * x1 + c1)
    v = fa * x0 + (fb * x1 + fc)

    r = jax.lax.round(v, jax.lax.RoundingMethod.TO_NEAREST_EVEN)
    sf = v - r
    t = sf * sf
    ct = jnp.float32(_COS_COEF[7])
    for k in (6, 5, 4, 3, 2, 1, 0):
        ct = ct * t + jnp.float32(_COS_COEF[k])
    base = jnp.maximum(hw_eps - ct * half_inv_w2, 1e-12)
    e1 = jnp.exp2(p * jnp.log2(base) + _C2)
    vals = jnp.exp2(-(u0 * u0 + (u1 * u1 + e1)))

    out_ref[:, :] = jnp.dot(vals, col_ref[:, :],
                            preferred_element_type=jnp.float32)


@jax.jit
def kernel(x, gaussian_means, gaussian_mats, subgaussian_frequency,
           subgaussian_offset, subgaussian_flat_top_power,
           subgaussian_width, subgaussian_rotation, colors):
    n_pix = x.shape[0]
    w = gaussian_means.shape[0]

    params = jnp.concatenate([
        gaussian_means[:, 0][None, :],
        gaussian_means[:, 1][None, :],
        gaussian_mats[:, 0, 0][None, :],
        gaussian_mats[:, 0, 1][None, :],
        gaussian_mats[:, 1, 0][None, :],
        gaussian_mats[:, 1, 1][None, :],
        subgaussian_rotation.T,
        subgaussian_frequency.T,
        subgaussian_offset.T,
        subgaussian_flat_top_power.T,
        subgaussian_width.T,
        jnp.zeros((5, w), jnp.float32),
    ], axis=0)

    grid = (n_pix // BLOCK_N,)
    return pl.pallas_call(
        _body,
        grid=grid,
        in_specs=[
            pl.BlockSpec((BLOCK_N, 2), lambda i: (i, 0)),
            pl.BlockSpec((16, w), lambda i: (0, 0)),
            pl.BlockSpec((w, N_CHANNELS), lambda i: (0, 0)),
        ],
        out_specs=pl.BlockSpec((BLOCK_N, N_CHANNELS), lambda i: (i, 0)),
        out_shape=jax.ShapeDtypeStruct((n_pix, N_CHANNELS), jnp.float32),
        compiler_params=pltpu.CompilerParams(
            dimension_semantics=("parallel",),
        ),
    )(x, params, colors)


# speedup vs baseline: 2.5144x; 1.3464x over previous
import jax
import jax.numpy as jnp
import numpy as np
from jax.experimental import pallas as pl
from jax.experimental.pallas import tpu as pltpu

N_CHANNELS = 3
BLOCK_N = 4096

_LOG2E = float(np.log2(np.e))
_KQ = -0.5 * _LOG2E
_C2 = float(np.log2(_LOG2E / 2.0))
_COS_COEF = (1.0 - 1.25e-6, -19.739202, 64.93908, -85.4497, 60.16561,
             -25.964163, 6.5281506)


def _body(xt_ref, pt_ref, colt_ref, out_ref):
    mx = pt_ref[:, 0:1]
    my = pt_ref[:, 1:2]
    m00 = pt_ref[:, 2:3]
    m01 = pt_ref[:, 3:4]
    m10 = pt_ref[:, 4:5]
    m11 = pt_ref[:, 5:6]
    rot = pt_ref[:, 6:7]
    freq = pt_ref[:, 7:8]
    off = pt_ref[:, 8:9]
    ftp = pt_ref[:, 9:10]
    logw = pt_ref[:, 10:11]

    kq = jnp.float32(_KQ)
    d0 = -(m00 * mx + m01 * my)
    d1 = -(m10 * mx + m11 * my)
    qa = kq * (m00 * m00 + m10 * m10)
    qb = (2.0 * kq) * (m00 * m01 + m10 * m11)
    qc = kq * (m01 * m01 + m11 * m11)
    qd = (2.0 * kq) * (m00 * d0 + m10 * d1)
    qe = (2.0 * kq) * (m01 * d0 + m11 * d1)

    c = jnp.cos(rot)
    s = jnp.sin(rot)
    f2 = 2.0 * freq
    fa = f2 * c
    fb = f2 * s
    fc = off * (1.0 / np.pi) - (fa * mx + fb * my)

    p = jnp.exp(ftp)
    kw = jnp.exp2(_C2 / p)
    hiw = (0.5 * kw) * jnp.exp(-2.0 * logw)
    hw_eps = hiw + 1e-12

    zeta = kq * (d0 * d0 + d1 * d1)
    colt = colt_ref[:, :] * jnp.exp2(zeta)

    x0 = xt_ref[0:1, :]
    x1 = xt_ref[1:2, :]
    x00 = x0 * x0
    x01 = x0 * x1
    x11 = x1 * x1

    nusq = (qa * x00 + qb * x01) + (qc * x11 + (qd * x0 + qe * x1))
    v = fa * x0 + (fb * x1 + fc)

    r = jax.lax.round(v, jax.lax.RoundingMethod.TO_NEAREST_EVEN)
    sf = v - r
    t = sf * sf
    ct = jnp.float32(_COS_COEF[6])
    for k in (5, 4, 3, 2, 1, 0):
        ct = ct * t + jnp.float32(_COS_COEF[k])
    base = hw_eps - ct * hiw
    e1 = jnp.exp2(p * jnp.log2(base))
    vals = jnp.exp2(nusq - e1)

    out_ref[:, :] = jax.lax.dot_general(
        colt, vals, (((0,), (0,)), ((), ())),
        preferred_element_type=jnp.float32)


@jax.jit
def kernel(x, gaussian_means, gaussian_mats, subgaussian_frequency,
           subgaussian_offset, subgaussian_flat_top_power,
           subgaussian_width, subgaussian_rotation, colors):
    n_pix = x.shape[0]
    w = gaussian_means.shape[0]

    params_t = jnp.concatenate([
        gaussian_means,
        gaussian_mats.reshape(w, 4),
        subgaussian_rotation,
        subgaussian_frequency,
        subgaussian_offset,
        subgaussian_flat_top_power,
        subgaussian_width,
        jnp.zeros((w, 5), jnp.float32),
    ], axis=1)
    xt = x.T

    out_t = pl.pallas_call(
        _body,
        grid=(n_pix // BLOCK_N,),
        in_specs=[
            pl.BlockSpec((2, BLOCK_N), lambda i: (0, i)),
            pl.BlockSpec((w, 16), lambda i: (0, 0)),
            pl.BlockSpec((w, N_CHANNELS), lambda i: (0, 0)),
        ],
        out_specs=pl.BlockSpec((N_CHANNELS, BLOCK_N), lambda i: (0, i)),
        out_shape=jax.ShapeDtypeStruct((N_CHANNELS, n_pix), jnp.float32),
        compiler_params=pltpu.CompilerParams(
            dimension_semantics=("parallel",),
        ),
    )(xt, params_t, colors)
    return out_t.T


# speedup vs baseline: 2.9752x; 1.1832x over previous
import jax
import jax.numpy as jnp
import numpy as np
from jax.experimental import pallas as pl
from jax.experimental.pallas import tpu as pltpu

N_CHANNELS = 3
BLOCK_N = 4096

_LOG2E = float(np.log2(np.e))
_KQ = -0.5 * _LOG2E
_C2 = float(np.log2(_LOG2E / 2.0))
_COS_COEF = (1.0 - 1.25e-6, -19.739202, 64.93908, -85.4497, 60.16561,
             -25.964163, 6.5281506)


def _coef_body(pt_ref, colt_ref, cf_ref, cs_ref):
    mx = pt_ref[:, 0:1]
    my = pt_ref[:, 1:2]
    m00 = pt_ref[:, 2:3]
    m01 = pt_ref[:, 3:4]
    m10 = pt_ref[:, 4:5]
    m11 = pt_ref[:, 5:6]
    rot = pt_ref[:, 6:7]
    freq = pt_ref[:, 7:8]
    off = pt_ref[:, 8:9]
    ftp = pt_ref[:, 9:10]
    logw = pt_ref[:, 10:11]

    kq = jnp.float32(_KQ)
    d0 = -(m00 * mx + m01 * my)
    d1 = -(m10 * mx + m11 * my)
    qa = kq * (m00 * m00 + m10 * m10)
    qb = (2.0 * kq) * (m00 * m01 + m10 * m11)
    qc = kq * (m01 * m01 + m11 * m11)
    qd = (2.0 * kq) * (m00 * d0 + m10 * d1)
    qe = (2.0 * kq) * (m01 * d0 + m11 * d1)

    c = jnp.cos(rot)
    s = jnp.sin(rot)
    f2 = 2.0 * freq
    fa = f2 * c
    fb = f2 * s
    fc = off * (1.0 / np.pi) - (fa * mx + fb * my)

    p = jnp.exp(ftp)
    kw = jnp.exp2(_C2 / p)
    hiw = (0.5 * kw) * jnp.exp(-2.0 * logw)
    hw_eps = hiw + 1e-12

    zero = jnp.zeros_like(mx)
    cf_ref[:, :] = jnp.concatenate(
        [qa, qb, qc, qd, qe, fa, fb, fc, hiw, hw_eps, p,
         zero, zero, zero, zero, zero], axis=1)

    zeta = kq * (d0 * d0 + d1 * d1)
    cs_ref[:, :] = colt_ref[:, :] * jnp.exp2(zeta)


def _main_body(xt_ref, cf_ref, cs_ref, out_ref):
    qa = cf_ref[:, 0:1]
    qb = cf_ref[:, 1:2]
    qc = cf_ref[:, 2:3]
    qd = cf_ref[:, 3:4]
    qe = cf_ref[:, 4:5]
    fa = cf_ref[:, 5:6]
    fb = cf_ref[:, 6:7]
    fc = cf_ref[:, 7:8]
    hiw = cf_ref[:, 8:9]
    hw_eps = cf_ref[:, 9:10]
    p = cf_ref[:, 10:11]

    x0 = xt_ref[0:1, :]
    x1 = xt_ref[1:2, :]
    x00 = x0 * x0
    x01 = x0 * x1
    x11 = x1 * x1

    nusq = (qa * x00 + qb * x01) + (qc * x11 + (qd * x0 + qe * x1))
    v = fa * x0 + (fb * x1 + fc)

    r = jax.lax.round(v, jax.lax.RoundingMethod.TO_NEAREST_EVEN)
    sf = v - r
    t = sf * sf
    ct = jnp.float32(_COS_COEF[6])
    for k in (5, 4, 3, 2, 1, 0):
        ct = ct * t + jnp.float32(_COS_COEF[k])
    base = hw_eps - ct * hiw
    e1 = jnp.exp2(p * jnp.log2(base))
    vals = jnp.exp2(nusq - e1)

    out_ref[:, :] = jax.lax.dot_general(
        cs_ref[:, :], vals, (((0,), (0,)), ((), ())),
        preferred_element_type=jnp.float32)


@jax.jit
def kernel(x, gaussian_means, gaussian_mats, subgaussian_frequency,
           subgaussian_offset, subgaussian_flat_top_power,
           subgaussian_width, subgaussian_rotation, colors):
    n_pix = x.shape[0]
    w = gaussian_means.shape[0]

    params_t = jnp.concatenate([
        gaussian_means,
        gaussian_mats.reshape(w, 4),
        subgaussian_rotation,
        subgaussian_frequency,
        subgaussian_offset,
        subgaussian_flat_top_power,
        subgaussian_width,
        jnp.zeros((w, 5), jnp.float32),
    ], axis=1)
    xt = x.T

    cf, cs = pl.pallas_call(
        _coef_body,
        in_specs=[
            pl.BlockSpec((w, 16), lambda: (0, 0)),
            pl.BlockSpec((w, N_CHANNELS), lambda: (0, 0)),
        ],
        out_specs=[
            pl.BlockSpec((w, 16), lambda: (0, 0)),
            pl.BlockSpec((w, N_CHANNELS), lambda: (0, 0)),
        ],
        out_shape=[
            jax.ShapeDtypeStruct((w, 16), jnp.float32),
            jax.ShapeDtypeStruct((w, N_CHANNELS), jnp.float32),
        ],
    )(params_t, colors)

    out_t = pl.pallas_call(
        _main_body,
        grid=(n_pix // BLOCK_N,),
        in_specs=[
            pl.BlockSpec((2, BLOCK_N), lambda i: (0, i)),
            pl.BlockSpec((w, 16), lambda i: (0, 0)),
            pl.BlockSpec((w, N_CHANNELS), lambda i: (0, 0)),
        ],
        out_specs=pl.BlockSpec((N_CHANNELS, BLOCK_N), lambda i: (0, i)),
        out_shape=jax.ShapeDtypeStruct((N_CHANNELS, n_pix), jnp.float32),
        compiler_params=pltpu.CompilerParams(
            dimension_semantics=("parallel",),
        ),
    )(xt, cf, cs)
    return out_t.T


# speedup vs baseline: 2.9944x; 1.0065x over previous
import jax
import jax.numpy as jnp
import numpy as np
from jax.experimental import pallas as pl
from jax.experimental.pallas import tpu as pltpu

N_CHANNELS = 3
BLOCK_N = 4096

_LOG2E = float(np.log2(np.e))
_KQ = -0.5 * _LOG2E
_C2 = float(np.log2(_LOG2E / 2.0))
_COS_COEF = (1.0 - 1.25e-6, -19.739202, 64.93908, -85.4497, 60.16561,
             -25.964163, 6.5281506)


def _coef_body(pt_ref, colt_ref, cf_ref, cs_ref):
    mx = pt_ref[:, 0:1]
    my = pt_ref[:, 1:2]
    m00 = pt_ref[:, 2:3]
    m01 = pt_ref[:, 3:4]
    m10 = pt_ref[:, 4:5]
    m11 = pt_ref[:, 5:6]
    rot = pt_ref[:, 6:7]
    freq = pt_ref[:, 7:8]
    off = pt_ref[:, 8:9]
    ftp = pt_ref[:, 9:10]
    logw = pt_ref[:, 10:11]

    kq = jnp.float32(_KQ)
    d0 = -(m00 * mx + m01 * my)
    d1 = -(m10 * mx + m11 * my)
    qa = kq * (m00 * m00 + m10 * m10)
    qb = (2.0 * kq) * (m00 * m01 + m10 * m11)
    qc = kq * (m01 * m01 + m11 * m11)
    qd = (2.0 * kq) * (m00 * d0 + m10 * d1)
    qe = (2.0 * kq) * (m01 * d0 + m11 * d1)

    c = jnp.cos(rot)
    s = jnp.sin(rot)
    f2 = 2.0 * freq
    fa = f2 * c
    fb = f2 * s
    fc = off * (1.0 / np.pi) - (fa * mx + fb * my)

    p = jnp.exp(ftp)
    kw = jnp.exp2(_C2 / p)
    hiw = (0.5 * kw) * jnp.exp(-2.0 * logw)
    hw_eps = hiw + 1e-12

    zero = jnp.zeros_like(mx)
    cf_ref[:, :] = jnp.concatenate(
        [qa, qb, qc, qd, qe, fa, fb, fc, hiw, hw_eps, p,
         zero, zero, zero, zero, zero], axis=1)

    zeta = kq * (d0 * d0 + d1 * d1)
    cs_ref[:, :] = colt_ref[:, :] * jnp.exp2(zeta)


def _main_body(xt_ref, cf_ref, cs_ref, out_ref):
    qa = cf_ref[:, 0:1]
    qb = cf_ref[:, 1:2]
    qc = cf_ref[:, 2:3]
    qd = cf_ref[:, 3:4]
    qe = cf_ref[:, 4:5]
    fa = cf_ref[:, 5:6]
    fb = cf_ref[:, 6:7]
    fc = cf_ref[:, 7:8]
    hiw = cf_ref[:, 8:9]
    hw_eps = cf_ref[:, 9:10]
    p = cf_ref[:, 10:11]

    x0 = xt_ref[0:1, :]
    x1 = xt_ref[1:2, :]

    nusq = (qa * x0 + (qb * x1 + qd)) * x0 + (qc * x1 + qe) * x1
    v = fa * x0 + (fb * x1 + fc)

    r = jax.lax.round(v, jax.lax.RoundingMethod.TO_NEAREST_EVEN)
    sf = v - r
    t = sf * sf
    ct = jnp.float32(_COS_COEF[6])
    for k in (5, 4, 3, 2, 1, 0):
        ct = ct * t + jnp.float32(_COS_COEF[k])
    base = hw_eps - ct * hiw
    e1 = jnp.exp2(p * jnp.log2(base))
    vals = jnp.exp2(nusq - e1)

    out_ref[:, :] = jax.lax.dot_general(
        cs_ref[:, :], vals, (((0,), (0,)), ((), ())),
        preferred_element_type=jnp.float32)


@jax.jit
def kernel(x, gaussian_means, gaussian_mats, subgaussian_frequency,
           subgaussian_offset, subgaussian_flat_top_power,
           subgaussian_width, subgaussian_rotation, colors):
    n_pix = x.shape[0]
    w = gaussian_means.shape[0]

    params_t = jnp.concatenate([
        gaussian_means,
        gaussian_mats.reshape(w, 4),
        subgaussian_rotation,
        subgaussian_frequency,
        subgaussian_offset,
        subgaussian_flat_top_power,
        subgaussian_width,
        jnp.zeros((w, 5), jnp.float32),
    ], axis=1)
    xt = x.T

    cf, cs = pl.pallas_call(
        _coef_body,
        in_specs=[
            pl.BlockSpec((w, 16), lambda: (0, 0)),
            pl.BlockSpec((w, N_CHANNELS), lambda: (0, 0)),
        ],
        out_specs=[
            pl.BlockSpec((w, 16), lambda: (0, 0)),
            pl.BlockSpec((w, N_CHANNELS), lambda: (0, 0)),
        ],
        out_shape=[
            jax.ShapeDtypeStruct((w, 16), jnp.float32),
            jax.ShapeDtypeStruct((w, N_CHANNELS), jnp.float32),
        ],
    )(params_t, colors)

    out_t = pl.pallas_call(
        _main_body,
        grid=(n_pix // BLOCK_N,),
        in_specs=[
            pl.BlockSpec((2, BLOCK_N), lambda i: (0, i)),
            pl.BlockSpec((w, 16), lambda i: (0, 0)),
            pl.BlockSpec((w, N_CHANNELS), lambda i: (0, 0)),
        ],
        out_specs=pl.BlockSpec((N_CHANNELS, BLOCK_N), lambda i: (0, i)),
        out_shape=jax.ShapeDtypeStruct((N_CHANNELS, n_pix), jnp.float32),
        compiler_params=pltpu.CompilerParams(
            dimension_semantics=("parallel",),
        ),
    )(xt, cf, cs)
    return out_t.T


# speedup vs baseline: 3.0730x; 1.0263x over previous
import jax
import jax.numpy as jnp
import numpy as np
from jax.experimental import pallas as pl
from jax.experimental.pallas import tpu as pltpu

N_CHANNELS = 3
BLOCK_N = 4096

_LOG2E = float(np.log2(np.e))
_KQ = -0.5 * _LOG2E
_C2 = float(np.log2(_LOG2E / 2.0))
_COS_COEF = (1.0 - 1.25e-6, -19.739202, 64.93908, -85.4497, 60.16561,
             -25.964163, 6.5281506)


def _coef_body(pt_ref, colt_ref, cf_ref, cs_ref):
    mx = pt_ref[:, 0:1]
    my = pt_ref[:, 1:2]
    m00 = pt_ref[:, 2:3]
    m01 = pt_ref[:, 3:4]
    m10 = pt_ref[:, 4:5]
    m11 = pt_ref[:, 5:6]
    rot = pt_ref[:, 6:7]
    freq = pt_ref[:, 7:8]
    off = pt_ref[:, 8:9]
    ftp = pt_ref[:, 9:10]
    logw = pt_ref[:, 10:11]

    kq = jnp.float32(_KQ)
    d0 = -(m00 * mx + m01 * my)
    d1 = -(m10 * mx + m11 * my)
    qa = kq * (m00 * m00 + m10 * m10)
    qb = (2.0 * kq) * (m00 * m01 + m10 * m11)
    qc = kq * (m01 * m01 + m11 * m11)
    qd = (2.0 * kq) * (m00 * d0 + m10 * d1)
    qe = (2.0 * kq) * (m01 * d0 + m11 * d1)

    c = jnp.cos(rot)
    s = jnp.sin(rot)
    f2 = 2.0 * freq
    fa = f2 * c
    fb = f2 * s
    fc = off * (1.0 / np.pi) - (fa * mx + fb * my)

    p = jnp.exp(ftp)
    kw = jnp.exp2(_C2 / p)
    p2 = p * jnp.float32(_LOG2E)
    hiw = (0.5 * kw) * jnp.exp(-2.0 * logw)
    hw_eps = hiw + 1e-12

    zero = jnp.zeros_like(mx)
    cf_ref[:, :] = jnp.concatenate(
        [qa, qb, qc, qd, qe, fa, fb, fc, hiw, hw_eps, p2,
         zero, zero, zero, zero, zero], axis=1)

    zeta = kq * (d0 * d0 + d1 * d1)
    cs_ref[:, :] = colt_ref[:, :] * jnp.exp2(zeta)


def _main_body(xt_ref, cf_ref, cs_ref, out_ref):
    qa = cf_ref[:, 0:1]
    qb = cf_ref[:, 1:2]
    qc = cf_ref[:, 2:3]
    qd = cf_ref[:, 3:4]
    qe = cf_ref[:, 4:5]
    fa = cf_ref[:, 5:6]
    fb = cf_ref[:, 6:7]
    fc = cf_ref[:, 7:8]
    hiw = cf_ref[:, 8:9]
    hw_eps = cf_ref[:, 9:10]
    p2 = cf_ref[:, 10:11]

    x0 = xt_ref[0:1, :]
    x1 = xt_ref[1:2, :]

    nusq = (qa * x0 + (qb * x1 + qd)) * x0 + (qc * x1 + qe) * x1
    v = fa * x0 + (fb * x1 + fc)

    r = jax.lax.round(v, jax.lax.RoundingMethod.TO_NEAREST_EVEN)
    sf = v - r
    t = sf * sf
    ct = jnp.float32(_COS_COEF[6])
    for k in (5, 4, 3, 2, 1, 0):
        ct = ct * t + jnp.float32(_COS_COEF[k])
    base = hw_eps - ct * hiw
    e1 = jnp.exp2(p2 * jnp.log(base))
    vals = jnp.exp2(nusq - e1)

    out_ref[:, :] = jax.lax.dot_general(
        cs_ref[:, :], vals, (((0,), (0,)), ((), ())),
        preferred_element_type=jnp.float32)


@jax.jit
def kernel(x, gaussian_means, gaussian_mats, subgaussian_frequency,
           subgaussian_offset, subgaussian_flat_top_power,
           subgaussian_width, subgaussian_rotation, colors):
    n_pix = x.shape[0]
    w = gaussian_means.shape[0]

    params_t = jnp.concatenate([
        gaussian_means,
        gaussian_mats.reshape(w, 4),
        subgaussian_rotation,
        subgaussian_frequency,
        subgaussian_offset,
        subgaussian_flat_top_power,
        subgaussian_width,
        jnp.zeros((w, 5), jnp.float32),
    ], axis=1)
    xt = x.T

    cf, cs = pl.pallas_call(
        _coef_body,
        in_specs=[
            pl.BlockSpec((w, 16), lambda: (0, 0)),
            pl.BlockSpec((w, N_CHANNELS), lambda: (0, 0)),
        ],
        out_specs=[
            pl.BlockSpec((w, 16), lambda: (0, 0)),
            pl.BlockSpec((w, N_CHANNELS), lambda: (0, 0)),
        ],
        out_shape=[
            jax.ShapeDtypeStruct((w, 16), jnp.float32),
            jax.ShapeDtypeStruct((w, N_CHANNELS), jnp.float32),
        ],
    )(params_t, colors)

    out_t = pl.pallas_call(
        _main_body,
        grid=(n_pix // BLOCK_N,),
        in_specs=[
            pl.BlockSpec((2, BLOCK_N), lambda i: (0, i)),
            pl.BlockSpec((w, 16), lambda i: (0, 0)),
            pl.BlockSpec((w, N_CHANNELS), lambda i: (0, 0)),
        ],
        out_specs=pl.BlockSpec((N_CHANNELS, BLOCK_N), lambda i: (0, i)),
        out_shape=jax.ShapeDtypeStruct((N_CHANNELS, n_pix), jnp.float32),
        compiler_params=pltpu.CompilerParams(
            dimension_semantics=("parallel",),
        ),
    )(xt, cf, cs)
    return out_t.T
